# Initial kernel scaffold; baseline (speedup 1.0000x reference)
import jax
import jax.numpy as jnp
from jax.experimental import pallas as pl


def kernel(q, r, ref_vals, tau, q_ln1_g, q_ln1_b, q_W, q_b, q_ln2_g, q_ln2_b, r_ln1_g, r_ln1_b, r_W, r_b, r_ln2_g, r_ln2_b, res_scale, scale):
    raise NotImplementedError("write your pallas kernel here")


# speedup vs baseline: 1.0804x; 1.0804x over previous
import functools
import math

import jax
import jax.numpy as jnp
from jax import lax
from jax.experimental import pallas as pl
from jax.experimental.pallas import tpu as pltpu

_B, _N, _D, _H, _K = 32, 8192, 128, 32, 256
_EPS = 1e-5


def _ln(x, g, b):
    m = jnp.mean(x, axis=-1, keepdims=True)
    xc = x - m
    v = jnp.mean(xc * xc, axis=-1, keepdims=True)
    return xc * lax.rsqrt(v + _EPS) * g + b


def _body(q_ref, r_ref, rv_ref, tau_ref,
          qg1_ref, qb1_ref, qW_ref, qb_ref, qg2_ref, qb2_ref,
          rg1_ref, rb1_ref, rW_ref, rb_ref, rg2_ref, rb2_ref,
          res_scale_ref, scale_ref,
          out_ref, scores_ref, qp_ref):
    i = pl.program_id(0)

    @pl.when(i == 0)
    def _():
        qn = _ln(q_ref[...], qg1_ref[...], qb1_ref[...])
        qv = jnp.dot(qn, qW_ref[...], preferred_element_type=jnp.float32,
                     precision=lax.Precision.HIGHEST) + qb_ref[...]
        qp_ref[...] = _ln(qv, qg2_ref[...], qb2_ref[...]) / math.sqrt(_H)

    x = r_ref[0]
    xn = _ln(x, rg1_ref[...], rb1_ref[...])
    v = jnp.dot(xn, rW_ref[...], preferred_element_type=jnp.float32,
                precision=lax.Precision.HIGHEST) + rb_ref[...]
    u = _ln(v, rg2_ref[...], rb2_ref[...])
    qpb = qp_ref[pl.ds(i, 1), :]
    s = jnp.sum(u * qpb, axis=-1)
    scores_ref[pl.ds(i, 1), :] = s.reshape(1, _N)

    @pl.when(i == _B - 1)
    def _():
        s2 = scores_ref[...]
        rv = rv_ref[...]
        tau = tau_ref[0, 0]
        smax = jnp.max(s2, axis=-1, keepdims=True)

        bits = lax.bitcast_convert_type(s2, jnp.int32)
        keys = jnp.where(bits < 0, bits ^ jnp.int32(0x7FFFFFFF), bits)

        lo0 = jnp.full((_B, 1), jnp.iinfo(jnp.int32).min, jnp.int32)
        hi0 = jnp.full((_B, 1), jnp.iinfo(jnp.int32).max, jnp.int32)

        def step(_, carry):
            lo, hi = carry
            mid = (lo & hi) + ((lo ^ hi) >> 1) + ((lo ^ hi) & 1)
            cnt = jnp.sum((keys >= mid).astype(jnp.int32), axis=-1,
                          keepdims=True)
            ge = cnt >= _K
            lo = jnp.where(ge, mid, lo)
            hi = jnp.where(ge, hi, mid - 1)
            return lo, hi

        t, _ = lax.fori_loop(0, 32, step, (lo0, hi0))

        gt = (keys > t).astype(jnp.float32)
        eq = (keys == t).astype(jnp.float32)
        c_gt = jnp.sum(gt, axis=-1, keepdims=True)
        c_eq = jnp.sum(eq, axis=-1, keepdims=True)
        e = jnp.exp((s2 - smax) / tau)
        frac = (_K - c_gt) / c_eq
        den = jnp.sum(e * gt, axis=-1, keepdims=True) + \
            frac * jnp.sum(e * eq, axis=-1, keepdims=True)
        num = jnp.sum(e * rv * gt, axis=-1, keepdims=True) + \
            frac * jnp.sum(e * rv * eq, axis=-1, keepdims=True)
        pred = num / den

        base = jnp.clip(pred, 0.0001, 1 - 0.0001)
        logit = jnp.log(base / (1.0 - base))
        z = scale_ref[0, 0] * logit + res_scale_ref[0, 0]
        out_ref[...] = 1.0 / (1.0 + jnp.exp(-z))


@jax.jit
def kernel(q, r, ref_vals, tau,
           q_ln1_g, q_ln1_b, q_W, q_b, q_ln2_g, q_ln2_b,
           r_ln1_g, r_ln1_b, r_W, r_b, r_ln2_g, r_ln2_b,
           res_scale, scale):
    row = lambda a: a.reshape(1, -1)
    one = lambda a: a.reshape(1, 1)
    const = lambda shape: pl.BlockSpec(shape, lambda i: (0,) * len(shape))

    out = pl.pallas_call(
        _body,
        grid=(_B,),
        in_specs=[
            const((_B, _D)),
            pl.BlockSpec((1, _N, _D), lambda i: (i, 0, 0)),
            const((_B, _N)),
            const((1, 1)),
            const((1, _D)), const((1, _D)),
            const((_D, _H)), const((1, _H)),
            const((1, _H)), const((1, _H)),
            const((1, _D)), const((1, _D)),
            const((_D, _H)), const((1, _H)),
            const((1, _H)), const((1, _H)),
            const((1, 1)), const((1, 1)),
        ],
        out_specs=const((_B, 1)),
        out_shape=jax.ShapeDtypeStruct((_B, 1), jnp.float32),
        scratch_shapes=[
            pltpu.VMEM((_B, _N), jnp.float32),
            pltpu.VMEM((_B, _H), jnp.float32),
        ],
    )(q, r, ref_vals, one(tau),
      row(q_ln1_g), row(q_ln1_b), q_W, row(q_b), row(q_ln2_g), row(q_ln2_b),
      row(r_ln1_g), row(r_ln1_b), r_W, row(r_b), row(r_ln2_g), row(r_ln2_b),
      one(res_scale), one(scale))
    return out.reshape(_B)


# speedup vs baseline: 3.8208x; 3.5366x over previous
import math

import jax
import jax.numpy as jnp
from jax import lax
from jax.experimental import pallas as pl
from jax.experimental.pallas import tpu as pltpu

_B, _N, _D, _H, _K = 32, 8192, 128, 32, 256
_EPS = 1e-5


def _ln(x, g, b):
    m = jnp.mean(x, axis=-1, keepdims=True)
    xc = x - m
    v = jnp.mean(xc * xc, axis=-1, keepdims=True)
    return xc * lax.rsqrt(v + _EPS) * g + b


def _dg(a, b, da, db):
    return lax.dot_general(a, b, (((da,), (db,)), ((), ())),
                           preferred_element_type=jnp.float32)


def _body(q_ref, r_ref, rv_ref, tau_ref,
          qg1_ref, qb1_ref, qW_ref, qb_ref, qg2_ref, qb2_ref,
          rg1_ref, rb1_ref, rW_ref, rb_ref, rg2_ref, rb2_ref,
          res_scale_ref, scale_ref,
          out_ref, scores_ref, cp_ref, c4_ref, wg_ref, scalb_ref, scalg_ref):
    i = pl.program_id(0)

    @pl.when(i == 0)
    def _():
        qn = _ln(q_ref[...], qg1_ref[...], qb1_ref[...])
        qv = jnp.dot(qn, qW_ref[...], preferred_element_type=jnp.float32,
                     precision=lax.Precision.HIGHEST) + qb_ref[...]
        qp = _ln(qv, qg2_ref[...], qb2_ref[...])
        wg = rW_ref[...] * rg1_ref[...]
        gw = jnp.sum(wg, axis=0, keepdims=True)
        bw = _dg(rb1_ref[...], rW_ref[...], 1, 0) + rb_ref[...]
        wg_ref[...] = wg
        p = qp * rg2_ref[...] * (1.0 / math.sqrt(_H))
        cp_ref[...] = _dg(p, wg, 1, 1)
        c4_ref[0:1, :] = _dg(jnp.ones((1, _H), jnp.float32), wg, 1, 1)
        c4_ref[1:2, :] = _dg(gw, wg, 1, 1)
        c4_ref[2:3, :] = _dg(bw, wg, 1, 1)
        c4_ref[3:4, :] = jnp.ones((1, _D), jnp.float32)
        c4_ref[4:8, :] = jnp.zeros((4, _D), jnp.float32)
        scalb_ref[:, 0:1] = jnp.sum(p * gw, axis=1, keepdims=True)
        scalb_ref[:, 1:2] = jnp.sum(p * bw, axis=1, keepdims=True)
        scalb_ref[:, 2:3] = jnp.sum(p, axis=1, keepdims=True)
        scalb_ref[:, 3:4] = jnp.sum(qp * rb2_ref[...], axis=1,
                                    keepdims=True) * (1.0 / math.sqrt(_H))
        scalb_ref[:, 4:8] = jnp.zeros((_B, 4), jnp.float32)
        scalg_ref[0:1, 0:1] = jnp.sum(gw, keepdims=True)
        scalg_ref[0:1, 1:2] = jnp.sum(bw, keepdims=True)
        scalg_ref[0:1, 2:3] = jnp.sum(gw * gw, keepdims=True)
        scalg_ref[0:1, 3:4] = jnp.sum(gw * bw, keepdims=True)
        scalg_ref[0:1, 4:5] = jnp.sum(bw * bw, keepdims=True)
        scalg_ref[0:1, 5:8] = jnp.zeros((1, 3), jnp.float32)

    x = r_ref[0]
    cp_b = cp_ref[pl.ds(i, 1), :]
    m_p = _dg(cp_b, x, 1, 1)
    m4 = _dg(c4_ref[0:4, :], x, 1, 1)
    s2 = _dg(jnp.ones((1, _D), jnp.float32), x * x, 1, 1)
    y = _dg(wg_ref[...], x, 0, 1)
    t2 = _dg(jnp.ones((1, _H), jnp.float32), y * y, 1, 0)

    m1 = m4[0:1, :]
    m_g = m4[1:2, :]
    m_b = m4[2:3, :]
    s1 = m4[3:4, :]

    sb = scalb_ref[pl.ds(i, 1), :]
    pgw_s, pbw_s, sp_s, qb2_s = (sb[:, 0:1], sb[:, 1:2], sb[:, 2:3],
                                 sb[:, 3:4])
    gl = scalg_ref[...]
    s_g, s_b, g2c, gbc, b2c = (gl[0:1, 0:1], gl[0:1, 1:2], gl[0:1, 2:3],
                               gl[0:1, 3:4], gl[0:1, 4:5])

    mu = s1 * (1.0 / _D)
    a = lax.rsqrt(s2 * (1.0 / _D) - mu * mu + _EPS)
    am = a * mu
    nu1 = a * m_p - am * pgw_s + pbw_s
    mu2 = (a * m1 - am * s_g + s_b) * (1.0 / _H)
    syc = m_b - am * m_g
    sc2 = am * am * g2c - 2.0 * am * gbc + b2c
    q2 = a * a * t2 + 2.0 * a * syc + sc2
    inv2 = lax.rsqrt(q2 * (1.0 / _H) - mu2 * mu2 + _EPS)
    scores_ref[pl.ds(i, 1), :] = inv2 * (nu1 - mu2 * sp_s) + qb2_s

    @pl.when(i == _B - 1)
    def _():
        s2d = scores_ref[...]
        rv = rv_ref[...]
        tau = tau_ref[0, 0]
        smax = jnp.max(s2d, axis=-1, keepdims=True)

        bits = lax.bitcast_convert_type(s2d, jnp.int32)
        keys = jnp.where(bits < 0, bits ^ jnp.int32(0x7FFFFFFF), bits)

        lo0 = jnp.full((_B, 1), jnp.iinfo(jnp.int32).min, jnp.int32)
        hi0 = jnp.full((_B, 1), jnp.iinfo(jnp.int32).max, jnp.int32)

        def step(_, carry):
            lo, hi = carry
            mid = (lo & hi) + ((lo ^ hi) >> 1) + ((lo ^ hi) & 1)
            cnt = jnp.sum((keys >= mid).astype(jnp.int32), axis=-1,
                          keepdims=True)
            ge = cnt >= _K
            lo = jnp.where(ge, mid, lo)
            hi = jnp.where(ge, hi, mid - 1)
            return lo, hi

        t, _ = lax.fori_loop(0, 32, step, (lo0, hi0))

        gt = (keys > t).astype(jnp.float32)
        eq = (keys == t).astype(jnp.float32)
        c_gt = jnp.sum(gt, axis=-1, keepdims=True)
        c_eq = jnp.sum(eq, axis=-1, keepdims=True)
        e = jnp.exp((s2d - smax) / tau)
        frac = (_K - c_gt) / c_eq
        den = jnp.sum(e * gt, axis=-1, keepdims=True) + \
            frac * jnp.sum(e * eq, axis=-1, keepdims=True)
        num = jnp.sum(e * rv * gt, axis=-1, keepdims=True) + \
            frac * jnp.sum(e * rv * eq, axis=-1, keepdims=True)
        pred = num / den

        base = jnp.clip(pred, 0.0001, 1 - 0.0001)
        logit = jnp.log(base / (1.0 - base))
        z = scale_ref[0, 0] * logit + res_scale_ref[0, 0]
        out_ref[...] = 1.0 / (1.0 + jnp.exp(-z))


@jax.jit
def kernel(q, r, ref_vals, tau,
           q_ln1_g, q_ln1_b, q_W, q_b, q_ln2_g, q_ln2_b,
           r_ln1_g, r_ln1_b, r_W, r_b, r_ln2_g, r_ln2_b,
           res_scale, scale):
    row = lambda a: a.reshape(1, -1)
    col = lambda a: a.reshape(-1, 1)
    one = lambda a: a.reshape(1, 1)
    const = lambda shape: pl.BlockSpec(shape, lambda i: (0,) * len(shape))

    out = pl.pallas_call(
        _body,
        grid=(_B,),
        in_specs=[
            const((_B, _D)),
            pl.BlockSpec((1, _N, _D), lambda i: (i, 0, 0)),
            const((_B, _N)),
            const((1, 1)),
            const((1, _D)), const((1, _D)),
            const((_D, _H)), const((1, _H)),
            const((1, _H)), const((1, _H)),
            const((_D, 1)), const((1, _D)),
            const((_D, _H)), const((1, _H)),
            const((1, _H)), const((1, _H)),
            const((1, 1)), const((1, 1)),
        ],
        out_specs=const((_B, 1)),
        out_shape=jax.ShapeDtypeStruct((_B, 1), jnp.float32),
        scratch_shapes=[
            pltpu.VMEM((_B, _N), jnp.float32),
            pltpu.VMEM((_B, _D), jnp.float32),
            pltpu.VMEM((8, _D), jnp.float32),
            pltpu.VMEM((_D, _H), jnp.float32),
            pltpu.VMEM((_B, 8), jnp.float32),
            pltpu.VMEM((1, 8), jnp.float32),
        ],
    )(q, r, ref_vals, one(tau),
      row(q_ln1_g), row(q_ln1_b), q_W, row(q_b), row(q_ln2_g), row(q_ln2_b),
      col(r_ln1_g), row(r_ln1_b), r_W, row(r_b), row(r_ln2_g), row(r_ln2_b),
      one(res_scale), one(scale))
    return out.reshape(_B)


# speedup vs baseline: 4.4285x; 1.1590x over previous
import math

import jax
import jax.numpy as jnp
from jax import lax
from jax.experimental import pallas as pl
from jax.experimental.pallas import tpu as pltpu

_B, _N, _D, _H, _K = 32, 8192, 128, 32, 256
_EPS = 1e-5


def _ln(x, g, b):
    m = jnp.mean(x, axis=-1, keepdims=True)
    xc = x - m
    v = jnp.mean(xc * xc, axis=-1, keepdims=True)
    return xc * lax.rsqrt(v + _EPS) * g + b


def _dg(a, b, da, db):
    return lax.dot_general(a, b, (((da,), (db,)), ((), ())),
                           preferred_element_type=jnp.float32)


def _body(q_ref, r_ref, rv_ref, tau_ref,
          qg1_ref, qb1_ref, qW_ref, qb_ref, qg2_ref, qb2_ref,
          rg1_ref, rb1_ref, rW_ref, rb_ref, rg2_ref, rb2_ref,
          res_scale_ref, scale_ref,
          out_ref, scores_ref, cp_ref, c4_ref, wg_ref, scalb_ref, scalg_ref):
    i = pl.program_id(0)

    @pl.when(i == 0)
    def _():
        qn = _ln(q_ref[...], qg1_ref[...], qb1_ref[...])
        qv = jnp.dot(qn, qW_ref[...], preferred_element_type=jnp.float32,
                     precision=lax.Precision.HIGHEST) + qb_ref[...]
        qp = _ln(qv, qg2_ref[...], qb2_ref[...])
        wg = rW_ref[...] * rg1_ref[...]
        gw = jnp.sum(wg, axis=0, keepdims=True)
        bw = _dg(rb1_ref[...], rW_ref[...], 1, 0) + rb_ref[...]
        wg_ref[...] = wg.astype(jnp.bfloat16)
        p = qp * rg2_ref[...] * (1.0 / math.sqrt(_H))
        cp_ref[...] = _dg(p, wg, 1, 1)
        c4_ref[0:1, :] = (_dg(jnp.ones((1, _H), jnp.float32), wg, 1, 1) *
                          (1.0 / _H)).astype(jnp.bfloat16)
        c4_ref[1:2, :] = (_dg(gw, wg, 1, 1) *
                          (2.0 / _H)).astype(jnp.bfloat16)
        c4_ref[2:3, :] = (_dg(bw, wg, 1, 1) *
                          (2.0 / _H)).astype(jnp.bfloat16)
        c4_ref[3:4, :] = jnp.full((1, _D), 1.0 / _D, jnp.bfloat16)
        c4_ref[4:8, :] = jnp.zeros((4, _D), jnp.bfloat16)
        scalb_ref[:, 0:1] = jnp.sum(p * gw, axis=1, keepdims=True)
        scalb_ref[:, 1:2] = jnp.sum(p * bw, axis=1, keepdims=True)
        scalb_ref[:, 2:3] = jnp.sum(p, axis=1, keepdims=True)
        scalb_ref[:, 3:4] = jnp.sum(qp * rb2_ref[...], axis=1,
                                    keepdims=True) * (1.0 / math.sqrt(_H))
        scalb_ref[:, 4:8] = jnp.zeros((_B, 4), jnp.float32)
        scalg_ref[0:1, 0:1] = jnp.sum(gw, keepdims=True) * (1.0 / _H)
        scalg_ref[0:1, 1:2] = jnp.sum(bw, keepdims=True) * (1.0 / _H)
        scalg_ref[0:1, 2:3] = jnp.sum(gw * gw, keepdims=True) * (1.0 / _H)
        scalg_ref[0:1, 3:4] = jnp.sum(gw * bw, keepdims=True) * (1.0 / _H)
        scalg_ref[0:1, 4:5] = jnp.sum(bw * bw, keepdims=True) * (1.0 / _H)
        scalg_ref[0:1, 5:8] = jnp.zeros((1, 3), jnp.float32)

    c4_ref[4:5, :] = cp_ref[pl.ds(i, 1), :].astype(jnp.bfloat16)
    xb = r_ref[0].astype(jnp.bfloat16)
    m5 = _dg(c4_ref[...], xb, 1, 1)
    s2 = _dg(jnp.full((1, _D), 1.0 / _D, jnp.bfloat16), xb * xb, 1, 1)
    y = _dg(wg_ref[...], xb, 0, 1)
    yb = y.astype(jnp.bfloat16)
    t2 = _dg(jnp.full((1, _H), 1.0 / _H, jnp.bfloat16), yb * yb, 1, 0)

    m1 = m5[0:1, :]
    m_g = m5[1:2, :]
    m_b = m5[2:3, :]
    mu = m5[3:4, :]
    m_p = m5[4:5, :]

    sb = scalb_ref[pl.ds(i, 1), :]
    pgw_s, pbw_s, sp_s, qb2_s = (sb[:, 0:1], sb[:, 1:2], sb[:, 2:3],
                                 sb[:, 3:4])
    gl = scalg_ref[...]
    s_g, s_b, g2c, gbc, b2c = (gl[0:1, 0:1], gl[0:1, 1:2], gl[0:1, 2:3],
                               gl[0:1, 3:4], gl[0:1, 4:5])

    a = lax.rsqrt(s2 - mu * mu + _EPS)
    am = a * mu
    nu1 = a * m_p - am * pgw_s + pbw_s
    mu2 = a * m1 - am * s_g + s_b
    syc = m_b - am * m_g
    sc2 = am * am * g2c - 2.0 * am * gbc + b2c
    q2 = a * a * t2 + a * syc + sc2
    inv2 = lax.rsqrt(q2 - mu2 * mu2 + _EPS)
    scores_ref[pl.ds(i, 1), :] = inv2 * (nu1 - mu2 * sp_s) + qb2_s

    @pl.when(i == _B - 1)
    def _():
        s2d = scores_ref[...]
        rv = rv_ref[...]
        tau = tau_ref[0, 0]
        smax = jnp.max(s2d, axis=-1, keepdims=True)

        bits = lax.bitcast_convert_type(s2d, jnp.int32)
        keys = jnp.where(bits < 0, bits ^ jnp.int32(0x7FFFFFFF), bits)

        lo0 = jnp.full((_B, 1), jnp.iinfo(jnp.int32).min, jnp.int32)
        hi0 = jnp.full((_B, 1), jnp.iinfo(jnp.int32).max, jnp.int32)

        def step(_, carry):
            lo, hi = carry
            mid = (lo & hi) + ((lo ^ hi) >> 1) + ((lo ^ hi) & 1)
            cnt = jnp.sum((keys >= mid).astype(jnp.int32), axis=-1,
                          keepdims=True)
            ge = cnt >= _K
            lo = jnp.where(ge, mid, lo)
            hi = jnp.where(ge, hi, mid - 1)
            return lo, hi

        t, _ = lax.fori_loop(0, 32, step, (lo0, hi0))

        gt = (keys > t).astype(jnp.float32)
        eq = (keys == t).astype(jnp.float32)
        c_gt = jnp.sum(gt, axis=-1, keepdims=True)
        c_eq = jnp.sum(eq, axis=-1, keepdims=True)
        e = jnp.exp((s2d - smax) / tau)
        frac = (_K - c_gt) / c_eq
        den = jnp.sum(e * gt, axis=-1, keepdims=True) + \
            frac * jnp.sum(e * eq, axis=-1, keepdims=True)
        num = jnp.sum(e * rv * gt, axis=-1, keepdims=True) + \
            frac * jnp.sum(e * rv * eq, axis=-1, keepdims=True)
        pred = num / den

        base = jnp.clip(pred, 0.0001, 1 - 0.0001)
        logit = jnp.log(base / (1.0 - base))
        z = scale_ref[0, 0] * logit + res_scale_ref[0, 0]
        out_ref[...] = 1.0 / (1.0 + jnp.exp(-z))


@jax.jit
def kernel(q, r, ref_vals, tau,
           q_ln1_g, q_ln1_b, q_W, q_b, q_ln2_g, q_ln2_b,
           r_ln1_g, r_ln1_b, r_W, r_b, r_ln2_g, r_ln2_b,
           res_scale, scale):
    row = lambda a: a.reshape(1, -1)
    col = lambda a: a.reshape(-1, 1)
    one = lambda a: a.reshape(1, 1)
    const = lambda shape: pl.BlockSpec(shape, lambda i: (0,) * len(shape))

    out = pl.pallas_call(
        _body,
        grid=(_B,),
        in_specs=[
            const((_B, _D)),
            pl.BlockSpec((1, _N, _D), lambda i: (i, 0, 0)),
            const((_B, _N)),
            const((1, 1)),
            const((1, _D)), const((1, _D)),
            const((_D, _H)), const((1, _H)),
            const((1, _H)), const((1, _H)),
            const((_D, 1)), const((1, _D)),
            const((_D, _H)), const((1, _H)),
            const((1, _H)), const((1, _H)),
            const((1, 1)), const((1, 1)),
        ],
        out_specs=const((_B, 1)),
        out_shape=jax.ShapeDtypeStruct((_B, 1), jnp.float32),
        scratch_shapes=[
            pltpu.VMEM((_B, _N), jnp.float32),
            pltpu.VMEM((_B, _D), jnp.float32),
            pltpu.VMEM((8, _D), jnp.bfloat16),
            pltpu.VMEM((_D, _H), jnp.bfloat16),
            pltpu.VMEM((_B, 8), jnp.float32),
            pltpu.VMEM((1, 8), jnp.float32),
        ],
    )(q, r, ref_vals, one(tau),
      row(q_ln1_g), row(q_ln1_b), q_W, row(q_b), row(q_ln2_g), row(q_ln2_b),
      col(r_ln1_g), row(r_ln1_b), r_W, row(r_b), row(r_ln2_g), row(r_ln2_b),
      one(res_scale), one(scale))
    return out.reshape(_B)


# speedup vs baseline: 4.9663x; 1.1214x over previous
import math

import jax
import jax.numpy as jnp
from jax import lax
from jax.experimental import pallas as pl
from jax.experimental.pallas import tpu as pltpu

_B, _N, _D, _H, _K = 32, 8192, 128, 32, 256
_EPS = 1e-5


def _ln(x, g, b):
    m = jnp.mean(x, axis=-1, keepdims=True)
    xc = x - m
    v = jnp.mean(xc * xc, axis=-1, keepdims=True)
    return xc * lax.rsqrt(v + _EPS) * g + b


def _dg(a, b, da, db):
    return lax.dot_general(a, b, (((da,), (db,)), ((), ())),
                           preferred_element_type=jnp.float32)


def _body(q_ref, r_ref, rv_ref, tau_ref,
          qg1_ref, qb1_ref, qW_ref, qb_ref, qg2_ref, qb2_ref,
          rg1_ref, rb1_ref, rW_ref, rb_ref, rg2_ref, rb2_ref,
          res_scale_ref, scale_ref,
          out_ref, scores_ref, cp_ref, c4_ref, scalb_ref, scalg_ref):
    i = pl.program_id(0)

    @pl.when(i == 0)
    def _():
        qn = _ln(q_ref[...], qg1_ref[...], qb1_ref[...])
        qv = jnp.dot(qn, qW_ref[...], preferred_element_type=jnp.float32,
                     precision=lax.Precision.HIGHEST) + qb_ref[...]
        qp = _ln(qv, qg2_ref[...], qb2_ref[...])
        wg = rW_ref[...] * rg1_ref[...]
        gw = jnp.sum(wg, axis=0, keepdims=True)
        bw = _dg(rb1_ref[...], rW_ref[...], 1, 0) + rb_ref[...]
        p = qp * rg2_ref[...] * (1.0 / math.sqrt(_H))
        cp_ref[...] = _dg(p, wg, 1, 1)
        c4_ref[0:1, :] = (_dg(jnp.ones((1, _H), jnp.float32), wg, 1, 1) *
                          (1.0 / _H)).astype(jnp.bfloat16)
        c4_ref[1:2, :] = (_dg(gw, wg, 1, 1) *
                          (2.0 / _H)).astype(jnp.bfloat16)
        c4_ref[2:3, :] = (_dg(bw, wg, 1, 1) *
                          (2.0 / _H)).astype(jnp.bfloat16)
        c4_ref[3:4, :] = jnp.full((1, _D), 1.0 / _D, jnp.bfloat16)
        c4_ref[4:8, :] = jnp.zeros((4, _D), jnp.bfloat16)
        c4_ref[8:40, :] = jnp.transpose(wg).astype(jnp.bfloat16)
        scalb_ref[:, 0:1] = jnp.sum(p * gw, axis=1, keepdims=True)
        scalb_ref[:, 1:2] = jnp.sum(p * bw, axis=1, keepdims=True)
        scalb_ref[:, 2:3] = jnp.sum(p, axis=1, keepdims=True)
        scalb_ref[:, 3:4] = jnp.sum(qp * rb2_ref[...], axis=1,
                                    keepdims=True) * (1.0 / math.sqrt(_H))
        scalb_ref[:, 4:8] = jnp.zeros((_B, 4), jnp.float32)
        scalg_ref[0:1, 0:1] = jnp.sum(gw, keepdims=True) * (1.0 / _H)
        scalg_ref[0:1, 1:2] = jnp.sum(bw, keepdims=True) * (1.0 / _H)
        scalg_ref[0:1, 2:3] = jnp.sum(gw * gw, keepdims=True) * (1.0 / _H)
        scalg_ref[0:1, 3:4] = jnp.sum(gw * bw, keepdims=True) * (1.0 / _H)
        scalg_ref[0:1, 4:5] = jnp.sum(bw * bw, keepdims=True) * (1.0 / _H)
        scalg_ref[0:1, 5:8] = jnp.zeros((1, 3), jnp.float32)

    c4_ref[4:5, :] = cp_ref[pl.ds(i, 1), :].astype(jnp.bfloat16)
    x = r_ref[0]
    xb = x.astype(jnp.bfloat16)
    m5 = _dg(c4_ref[...], xb, 1, 1)
    s2 = _dg(jnp.full((1, _D), 1.0 / _D, jnp.float32), x * x, 1, 1)
    y = m5[8:40, :]
    t2 = _dg(jnp.full((1, _H), 1.0 / _H, jnp.float32), y * y, 1, 0)

    m1 = m5[0:1, :]
    m_g = m5[1:2, :]
    m_b = m5[2:3, :]
    mu = m5[3:4, :]
    m_p = m5[4:5, :]

    sb = scalb_ref[pl.ds(i, 1), :]
    pgw_s, pbw_s, sp_s, qb2_s = (sb[:, 0:1], sb[:, 1:2], sb[:, 2:3],
                                 sb[:, 3:4])
    gl = scalg_ref[...]
    s_g, s_b, g2c, gbc, b2c = (gl[0:1, 0:1], gl[0:1, 1:2], gl[0:1, 2:3],
                               gl[0:1, 3:4], gl[0:1, 4:5])

    a = lax.rsqrt(s2 - mu * mu + _EPS)
    am = a * mu
    nu1 = a * m_p - am * pgw_s + pbw_s
    mu2 = a * m1 - am * s_g + s_b
    syc = m_b - am * m_g
    sc2 = am * am * g2c - 2.0 * am * gbc + b2c
    q2 = a * a * t2 + a * syc + sc2
    inv2 = lax.rsqrt(q2 - mu2 * mu2 + _EPS)
    scores_ref[pl.ds(i, 1), :] = inv2 * (nu1 - mu2 * sp_s) + qb2_s

    @pl.when(i == _B - 1)
    def _():
        s2d = scores_ref[...]
        rv = rv_ref[...]
        tau = tau_ref[0, 0]
        smax = jnp.max(s2d, axis=-1, keepdims=True)

        bits = lax.bitcast_convert_type(s2d, jnp.int32)
        keys = jnp.where(bits < 0, bits ^ jnp.int32(0x7FFFFFFF), bits)

        lo0 = jnp.full((_B, 1), jnp.iinfo(jnp.int32).min, jnp.int32)
        hi0 = jnp.full((_B, 1), jnp.iinfo(jnp.int32).max, jnp.int32)

        def step(_, carry):
            lo, hi = carry
            mid = (lo & hi) + ((lo ^ hi) >> 1) + ((lo ^ hi) & 1)
            cnt = jnp.sum((keys >= mid).astype(jnp.int32), axis=-1,
                          keepdims=True)
            ge = cnt >= _K
            lo = jnp.where(ge, mid, lo)
            hi = jnp.where(ge, hi, mid - 1)
            return lo, hi

        t, _ = lax.fori_loop(0, 32, step, (lo0, hi0))

        gt = (keys > t).astype(jnp.float32)
        eq = (keys == t).astype(jnp.float32)
        c_gt = jnp.sum(gt, axis=-1, keepdims=True)
        c_eq = jnp.sum(eq, axis=-1, keepdims=True)
        e = jnp.exp((s2d - smax) / tau)
        frac = (_K - c_gt) / c_eq
        den = jnp.sum(e * gt, axis=-1, keepdims=True) + \
            frac * jnp.sum(e * eq, axis=-1, keepdims=True)
        num = jnp.sum(e * rv * gt, axis=-1, keepdims=True) + \
            frac * jnp.sum(e * rv * eq, axis=-1, keepdims=True)
        pred = num / den

        base = jnp.clip(pred, 0.0001, 1 - 0.0001)
        logit = jnp.log(base / (1.0 - base))
        z = scale_ref[0, 0] * logit + res_scale_ref[0, 0]
        out_ref[...] = 1.0 / (1.0 + jnp.exp(-z))


@jax.jit
def kernel(q, r, ref_vals, tau,
           q_ln1_g, q_ln1_b, q_W, q_b, q_ln2_g, q_ln2_b,
           r_ln1_g, r_ln1_b, r_W, r_b, r_ln2_g, r_ln2_b,
           res_scale, scale):
    row = lambda a: a.reshape(1, -1)
    col = lambda a: a.reshape(-1, 1)
    one = lambda a: a.reshape(1, 1)
    const = lambda shape: pl.BlockSpec(shape, lambda i: (0,) * len(shape))

    out = pl.pallas_call(
        _body,
        grid=(_B,),
        in_specs=[
            const((_B, _D)),
            pl.BlockSpec((1, _N, _D), lambda i: (i, 0, 0)),
            const((_B, _N)),
            const((1, 1)),
            const((1, _D)), const((1, _D)),
            const((_D, _H)), const((1, _H)),
            const((1, _H)), const((1, _H)),
            const((_D, 1)), const((1, _D)),
            const((_D, _H)), const((1, _H)),
            const((1, _H)), const((1, _H)),
            const((1, 1)), const((1, 1)),
        ],
        out_specs=const((_B, 1)),
        out_shape=jax.ShapeDtypeStruct((_B, 1), jnp.float32),
        scratch_shapes=[
            pltpu.VMEM((_B, _N), jnp.float32),
            pltpu.VMEM((_B, _D), jnp.float32),
            pltpu.VMEM((48, _D), jnp.bfloat16),
            pltpu.VMEM((_B, 8), jnp.float32),
            pltpu.VMEM((1, 8), jnp.float32),
        ],
    )(q, r, ref_vals, one(tau),
      row(q_ln1_g), row(q_ln1_b), q_W, row(q_b), row(q_ln2_g), row(q_ln2_b),
      col(r_ln1_g), row(r_ln1_b), r_W, row(r_b), row(r_ln2_g), row(r_ln2_b),
      one(res_scale), one(scale))
    return out.reshape(_B)
